# baseline (speedup 1.0000x reference)
import functools

import jax
import jax.numpy as jnp
from jax import lax
from jax.experimental import pallas as pl
from jax.experimental.pallas import tpu as pltpu
from jax.experimental.pallas import tpu_sc as plsc

N_A = 16384
N_B = 16384
D = 256
TOPK = 3
FULL_SCALE = 128
R = 0.5

BLK_A = 256
KBLK_A = 1024
KGRID_A = N_A // KBLK_A
BLK_B = 2048
N_BLK_B = N_B // BLK_B
GRID_A = N_A // BLK_A
BIG = (1 << 30) + (1 << 24)

SC_WORKERS = 32
SC_TOTAL = N_A * TOPK
SC_PER_W = SC_TOTAL // SC_WORKERS
SC_CHUNK = 128
SC_N_CHUNK = SC_PER_W // SC_CHUNK


def _knn_body(qa_ref, ga_ref, qb_ref, gb_ref, idx_ref, w_ref, seg_ref):
    qa = qa_ref[...]
    ga = jnp.min(ga_ref[...], axis=1, keepdims=True)
    an = jnp.sum(qa * qa, axis=1, keepdims=True)
    iota = (lax.broadcasted_iota(jnp.int32, (KBLK_A, BLK_B), 1)
            + (1 << 23))
    big = jnp.int32(BIG)

    @pl.when(pl.program_id(0) == 0)
    def _():
        gb = gb_ref[...]
        for g in range(5):
            cnt = jnp.sum((gb < g).astype(jnp.int32))
            seg_ref[g] = cnt // 8

    g_lo = jnp.min(ga)
    g_hi = jnp.max(ga)
    t_lo = seg_ref[g_lo] // BLK_B
    t_hi = (seg_ref[g_hi + 1] + BLK_B - 1) // BLK_B

    r0 = jnp.full((KBLK_A, 1), big, jnp.int32)

    def scan_block(t, carry):
        r0, r1, r2 = carry
        base = t * BLK_B
        qb = qb_ref[:, pl.ds(base, BLK_B)]
        xy = jnp.dot(qa, qb, preferred_element_type=jnp.float32)
        bn = jnp.sum(qb * qb, axis=0, keepdims=True)
        d2 = (an + bn) - 2.0 * xy
        d2i = jnp.minimum(d2, 65536.0).astype(jnp.int32)
        keys = d2i * 16384 + (iota + base)

        kf = keys
        h = BLK_B // 2
        p0 = jnp.minimum(kf[:, 0:h], kf[:, h:BLK_B])
        p1 = jnp.maximum(kf[:, 0:h], kf[:, h:BLK_B])
        h //= 2
        a0, b0 = p0[:, 0:h], p0[:, h:2 * h]
        a1, b1 = p1[:, 0:h], p1[:, h:2 * h]
        t0 = jnp.minimum(a0, b0)
        x = jnp.maximum(a0, b0)
        y = jnp.minimum(a1, b1)
        t1 = jnp.minimum(x, y)
        t2 = jnp.maximum(x, y)

        def tri_merge(u0, u1, u2):
            h = u0.shape[1] // 2
            a0, b0 = u0[:, :h], u0[:, h:]
            a1, b1 = u1[:, :h], u1[:, h:]
            a2, b2 = u2[:, :h], u2[:, h:]
            m0 = jnp.minimum(a0, b0)
            hi = jnp.maximum(a0, b0)
            lo = jnp.minimum(a1, b1)
            m1 = jnp.minimum(hi, lo)
            tt = jnp.maximum(hi, lo)
            m2 = jnp.minimum(tt, jnp.minimum(a2, b2))
            return m0, m1, m2

        while h > 128:
            t0, t1, t2 = tri_merge(t0, t1, t2)
            h //= 2

        ks = jnp.concatenate([t0, t1, t2], axis=1)
        for _ in range(TOPK):
            m = jnp.min(ks, axis=1, keepdims=True)
            ks = jnp.where(ks == m, big, ks)
            h0 = jnp.maximum(r0, m)
            r0 = jnp.minimum(r0, m)
            h1 = jnp.maximum(r1, h0)
            r1 = jnp.minimum(r1, h0)
            r2 = jnp.minimum(r2, h1)
        return r0, r1, r2

    r0, r1, r2 = lax.fori_loop(t_lo, t_hi, scan_block, (r0, r0, r0))

    rr = jnp.concatenate([r0, r1, r2], axis=1)
    d2f = ((rr >> 14) - 512).astype(jnp.float32)
    dist = jnp.sqrt(d2f) * (1.0 / FULL_SCALE)
    idx_ref[...] = rr & 16383
    w_ref[...] = jnp.maximum(0.0, R - dist)


def _knn(qa, ga_pad, qb, gb_pad):
    return pl.pallas_call(
        _knn_body,
        grid=(KGRID_A,),
        in_specs=[
            pl.BlockSpec((KBLK_A, 8), lambda i: (i, 0)),
            pl.BlockSpec((KBLK_A, 8), lambda i: (i, 0)),
            pl.BlockSpec((8, N_B), lambda i: (0, 0)),
            pl.BlockSpec((8, N_B), lambda i: (0, 0)),
        ],
        out_specs=[
            pl.BlockSpec((KBLK_A, TOPK), lambda i: (i, 0)),
            pl.BlockSpec((KBLK_A, TOPK), lambda i: (i, 0)),
        ],
        out_shape=[
            jax.ShapeDtypeStruct((N_A, TOPK), jnp.int32),
            jax.ShapeDtypeStruct((N_A, TOPK), jnp.float32),
        ],
        scratch_shapes=[pltpu.SMEM((8,), jnp.int32)],
    )(qa, ga_pad, qb, gb_pad)


@functools.cache
def _build_sc_gather():
    mesh = plsc.VectorSubcoreMesh(core_axis_name="c", subcore_axis_name="s")

    @functools.partial(
        pl.kernel,
        mesh=mesh,
        out_type=jax.ShapeDtypeStruct((SC_TOTAL, D), jnp.float32),
        scratch_types=[
            pltpu.VMEM((SC_PER_W,), jnp.int32),
            pltpu.VMEM((SC_CHUNK, D), jnp.float32),
            pltpu.VMEM((SC_CHUNK, D), jnp.float32),
            pltpu.SemaphoreType.DMA,
            pltpu.SemaphoreType.DMA,
            pltpu.SemaphoreType.DMA,
            pltpu.SemaphoreType.DMA,
        ],
    )
    def sc_gather(table_hbm, idx_hbm, out_hbm, idx_v, rv0, rv1,
                  gs0, gs1, ws0, ws1):
        wid = lax.axis_index("s") * 2 + lax.axis_index("c")
        base = wid * SC_PER_W
        pltpu.sync_copy(idx_hbm.at[pl.ds(base, SC_PER_W)], idx_v)
        bufs = (rv0, rv1)
        gsem = (gs0, gs1)
        wsem = (ws0, ws1)

        def start_gather(c):
            return pltpu.async_copy(
                table_hbm.at[idx_v.at[pl.ds(c * SC_CHUNK, SC_CHUNK)]],
                bufs[c % 2], gsem[c % 2])

        h_g = start_gather(0)
        h_w = [None] * SC_N_CHUNK
        for c in range(SC_N_CHUNK):
            if c + 1 < SC_N_CHUNK:
                if c >= 1:
                    h_w[c - 1].wait()
                h_g_next = start_gather(c + 1)
            h_g.wait()
            h_w[c] = pltpu.async_copy(
                bufs[c % 2],
                out_hbm.at[pl.ds(base + c * SC_CHUNK, SC_CHUNK)],
                wsem[c % 2])
            if c + 1 < SC_N_CHUNK:
                h_g = h_g_next
        h_w[SC_N_CHUNK - 2].wait()
        h_w[SC_N_CHUNK - 1].wait()

    return sc_gather


def _combine_body(aF_ref, W_ref, bfuse_ref, rows_ref, w_ref, out_ref):
    W1 = W_ref[0:D, :]
    W2 = W_ref[D:2 * D, :]
    W12 = W1 - W2
    aF = aF_ref[...]
    A = jnp.dot(aF, W2, preferred_element_type=jnp.float32) + bfuse_ref[...]
    acc = jnp.zeros((BLK_A, D), jnp.float32)
    for k in range(TOPK):
        mk = jnp.dot(rows_ref[k], W12, preferred_element_type=jnp.float32)
        acc = acc + jnp.maximum(A + mk, 0.0) * w_ref[:, k:k + 1]
    out_ref[:, 0:D] = aF
    out_ref[:, D:2 * D] = acc


def _combine(a_F, W_fuse, b_fuse, rows3, w):
    return pl.pallas_call(
        _combine_body,
        grid=(GRID_A,),
        in_specs=[
            pl.BlockSpec((BLK_A, D), lambda i: (i, 0)),
            pl.BlockSpec((2 * D, D), lambda i: (0, 0)),
            pl.BlockSpec((1, D), lambda i: (0, 0)),
            pl.BlockSpec((TOPK, BLK_A, D), lambda i: (0, i, 0)),
            pl.BlockSpec((BLK_A, TOPK), lambda i: (i, 0)),
        ],
        out_specs=pl.BlockSpec((BLK_A, 2 * D), lambda i: (i, 0)),
        out_shape=jax.ShapeDtypeStruct((N_A, 2 * D), jnp.float32),
    )(a_F, W_fuse, b_fuse.reshape(1, D), rows3, w)


def _prep(point_idx_a, coord_a, point_idx_b, coord_b):
    ca = coord_a.astype(jnp.float32)
    cb = coord_b.astype(jnp.float32)
    pa = (point_idx_a.astype(jnp.float32) * 221.0)[:, None]
    pb = (point_idx_b.astype(jnp.float32) * 221.0)[:, None]
    qa = jnp.pad(jnp.concatenate([ca, pa], axis=1), ((0, 0), (0, 4)))
    qb = jnp.pad(jnp.concatenate([cb, pb], axis=1), ((0, 0), (0, 4))).T
    ga_pad = jnp.broadcast_to(point_idx_a[:, None].astype(jnp.int32),
                              (N_A, 8))
    gb_pad = jnp.broadcast_to(point_idx_b[None, :].astype(jnp.int32),
                              (8, N_B))
    return qa, ga_pad, qb, gb_pad


def kernel(point_idx_a, coord_a, a_F, point_idx_b, coord_b, b_F,
           W_fuse, b_fuse):
    qa, ga_pad, qb, gb_pad = _prep(point_idx_a, coord_a,
                                   point_idx_b, coord_b)

    idx, w = _knn(qa, ga_pad, qb, gb_pad)
    rows = _build_sc_gather()(b_F, idx.T.reshape(-1))
    return _combine(a_F, W_fuse, b_fuse, rows.reshape(TOPK, N_A, D), w)


# speedup vs baseline: 1.0611x; 1.0611x over previous
import functools

import jax
import jax.numpy as jnp
from jax import lax
from jax.experimental import pallas as pl
from jax.experimental.pallas import tpu as pltpu
from jax.experimental.pallas import tpu_sc as plsc

N_A = 16384
N_B = 16384
D = 256
TOPK = 3
FULL_SCALE = 128
R = 0.5

BLK_A = 512
KBLK_A = 512
KGRID_A = N_A // KBLK_A
BLK_B = 2048
N_BLK_B = N_B // BLK_B
GRID_A = N_A // BLK_A
BIG = (1 << 30) + (1 << 24)

SC_WORKERS = 32
SC_TOTAL = N_A * TOPK
SC_PER_W = SC_TOTAL // SC_WORKERS
SC_CHUNK = 128
SC_N_CHUNK = SC_PER_W // SC_CHUNK


def _knn_body(qa_ref, ga_ref, qb_ref, gb_ref, idx_ref, w_ref, seg_ref):
    qa = qa_ref[...]
    ga = jnp.min(ga_ref[...], axis=1, keepdims=True)
    an = jnp.sum(qa * qa, axis=1, keepdims=True)
    iota = (lax.broadcasted_iota(jnp.int32, (KBLK_A, BLK_B), 1)
            + (1 << 23))
    big = jnp.int32(BIG)

    @pl.when(pl.program_id(0) == 0)
    def _():
        gb = gb_ref[...]
        for g in range(5):
            cnt = jnp.sum((gb < g).astype(jnp.int32))
            seg_ref[g] = cnt // 8

    g_lo = jnp.min(ga)
    g_hi = jnp.max(ga)
    t_lo = seg_ref[g_lo] // BLK_B
    t_hi = (seg_ref[g_hi + 1] + BLK_B - 1) // BLK_B

    r0 = jnp.full((KBLK_A, 1), big, jnp.int32)

    def scan_block(t, carry):
        r0, r1, r2 = carry
        base = t * BLK_B
        qb = qb_ref[:, pl.ds(base, BLK_B)]
        xy = jnp.dot(qa, qb, preferred_element_type=jnp.float32)
        bn = jnp.sum(qb * qb, axis=0, keepdims=True)
        d2 = (an + bn) - 2.0 * xy
        d2i = jnp.minimum(d2, 65536.0).astype(jnp.int32)
        keys = d2i * 16384 + (iota + base)

        kf = keys
        h = BLK_B // 2
        p0 = jnp.minimum(kf[:, 0:h], kf[:, h:BLK_B])
        p1 = jnp.maximum(kf[:, 0:h], kf[:, h:BLK_B])
        h //= 2
        a0, b0 = p0[:, 0:h], p0[:, h:2 * h]
        a1, b1 = p1[:, 0:h], p1[:, h:2 * h]
        t0 = jnp.minimum(a0, b0)
        x = jnp.maximum(a0, b0)
        y = jnp.minimum(a1, b1)
        t1 = jnp.minimum(x, y)
        t2 = jnp.maximum(x, y)

        def tri_merge(u0, u1, u2):
            h = u0.shape[1] // 2
            a0, b0 = u0[:, :h], u0[:, h:]
            a1, b1 = u1[:, :h], u1[:, h:]
            a2, b2 = u2[:, :h], u2[:, h:]
            m0 = jnp.minimum(a0, b0)
            hi = jnp.maximum(a0, b0)
            lo = jnp.minimum(a1, b1)
            m1 = jnp.minimum(hi, lo)
            tt = jnp.maximum(hi, lo)
            m2 = jnp.minimum(tt, jnp.minimum(a2, b2))
            return m0, m1, m2

        while h > 128:
            t0, t1, t2 = tri_merge(t0, t1, t2)
            h //= 2

        ks = jnp.concatenate([t0, t1, t2], axis=1)
        for _ in range(TOPK):
            m = jnp.min(ks, axis=1, keepdims=True)
            ks = jnp.where(ks == m, big, ks)
            h0 = jnp.maximum(r0, m)
            r0 = jnp.minimum(r0, m)
            h1 = jnp.maximum(r1, h0)
            r1 = jnp.minimum(r1, h0)
            r2 = jnp.minimum(r2, h1)
        return r0, r1, r2

    r0, r1, r2 = lax.fori_loop(t_lo, t_hi, scan_block, (r0, r0, r0))

    rr = jnp.concatenate([r0, r1, r2], axis=1)
    d2f = ((rr >> 14) - 512).astype(jnp.float32)
    dist = jnp.sqrt(d2f) * (1.0 / FULL_SCALE)
    idx_ref[...] = rr & 16383
    w_ref[...] = jnp.maximum(0.0, R - dist)


def _knn(qa, ga_pad, qb, gb_pad):
    return pl.pallas_call(
        _knn_body,
        grid=(KGRID_A,),
        in_specs=[
            pl.BlockSpec((KBLK_A, 8), lambda i: (i, 0)),
            pl.BlockSpec((KBLK_A, 8), lambda i: (i, 0)),
            pl.BlockSpec((8, N_B), lambda i: (0, 0)),
            pl.BlockSpec((8, N_B), lambda i: (0, 0)),
        ],
        out_specs=[
            pl.BlockSpec((KBLK_A, TOPK), lambda i: (i, 0)),
            pl.BlockSpec((KBLK_A, TOPK), lambda i: (i, 0)),
        ],
        out_shape=[
            jax.ShapeDtypeStruct((N_A, TOPK), jnp.int32),
            jax.ShapeDtypeStruct((N_A, TOPK), jnp.float32),
        ],
        scratch_shapes=[pltpu.SMEM((8,), jnp.int32)],
    )(qa, ga_pad, qb, gb_pad)


@functools.cache
def _build_sc_gather():
    mesh = plsc.VectorSubcoreMesh(core_axis_name="c", subcore_axis_name="s")

    @functools.partial(
        pl.kernel,
        mesh=mesh,
        out_type=jax.ShapeDtypeStruct((SC_TOTAL, D), jnp.float32),
        scratch_types=[
            pltpu.VMEM((SC_PER_W,), jnp.int32),
            pltpu.VMEM((SC_CHUNK, D), jnp.float32),
            pltpu.VMEM((SC_CHUNK, D), jnp.float32),
            pltpu.SemaphoreType.DMA,
            pltpu.SemaphoreType.DMA,
            pltpu.SemaphoreType.DMA,
            pltpu.SemaphoreType.DMA,
        ],
    )
    def sc_gather(table_hbm, idx_hbm, out_hbm, idx_v, rv0, rv1,
                  gs0, gs1, ws0, ws1):
        wid = lax.axis_index("s") * 2 + lax.axis_index("c")
        base = wid * SC_PER_W
        pltpu.sync_copy(idx_hbm.at[pl.ds(base, SC_PER_W)], idx_v)
        bufs = (rv0, rv1)
        gsem = (gs0, gs1)
        wsem = (ws0, ws1)

        def start_gather(c):
            return pltpu.async_copy(
                table_hbm.at[idx_v.at[pl.ds(c * SC_CHUNK, SC_CHUNK)]],
                bufs[c % 2], gsem[c % 2])

        h_g = start_gather(0)
        h_w = [None] * SC_N_CHUNK
        for c in range(SC_N_CHUNK):
            if c + 1 < SC_N_CHUNK:
                if c >= 1:
                    h_w[c - 1].wait()
                h_g_next = start_gather(c + 1)
            h_g.wait()
            h_w[c] = pltpu.async_copy(
                bufs[c % 2],
                out_hbm.at[pl.ds(base + c * SC_CHUNK, SC_CHUNK)],
                wsem[c % 2])
            if c + 1 < SC_N_CHUNK:
                h_g = h_g_next
        h_w[SC_N_CHUNK - 2].wait()
        h_w[SC_N_CHUNK - 1].wait()

    return sc_gather


def _combine_body(aF_ref, W_ref, bfuse_ref, rows_ref, w_ref, out_ref):
    W1 = W_ref[0:D, :]
    W2 = W_ref[D:2 * D, :]
    W12 = W1 - W2
    aF = aF_ref[...]
    A = jnp.dot(aF, W2, preferred_element_type=jnp.float32) + bfuse_ref[...]
    acc = jnp.zeros((BLK_A, D), jnp.float32)
    for k in range(TOPK):
        mk = jnp.dot(rows_ref[k], W12, preferred_element_type=jnp.float32)
        acc = acc + jnp.maximum(A + mk, 0.0) * w_ref[:, k:k + 1]
    out_ref[:, 0:D] = aF
    out_ref[:, D:2 * D] = acc


def _combine(a_F, W_fuse, b_fuse, rows3, w):
    return pl.pallas_call(
        _combine_body,
        grid=(GRID_A,),
        in_specs=[
            pl.BlockSpec((BLK_A, D), lambda i: (i, 0)),
            pl.BlockSpec((2 * D, D), lambda i: (0, 0)),
            pl.BlockSpec((1, D), lambda i: (0, 0)),
            pl.BlockSpec((TOPK, BLK_A, D), lambda i: (0, i, 0)),
            pl.BlockSpec((BLK_A, TOPK), lambda i: (i, 0)),
        ],
        out_specs=pl.BlockSpec((BLK_A, 2 * D), lambda i: (i, 0)),
        out_shape=jax.ShapeDtypeStruct((N_A, 2 * D), jnp.float32),
    )(a_F, W_fuse, b_fuse.reshape(1, D), rows3, w)


def _prep(point_idx_a, coord_a, point_idx_b, coord_b):
    ca = coord_a.astype(jnp.float32)
    cb = coord_b.astype(jnp.float32)
    pa = (point_idx_a.astype(jnp.float32) * 221.0)[:, None]
    pb = (point_idx_b.astype(jnp.float32) * 221.0)[:, None]
    qa = jnp.pad(jnp.concatenate([ca, pa], axis=1), ((0, 0), (0, 4)))
    qb = jnp.pad(jnp.concatenate([cb, pb], axis=1), ((0, 0), (0, 4))).T
    ga_pad = jnp.broadcast_to(point_idx_a[:, None].astype(jnp.int32),
                              (N_A, 8))
    gb_pad = jnp.broadcast_to(point_idx_b[None, :].astype(jnp.int32),
                              (8, N_B))
    return qa, ga_pad, qb, gb_pad


def kernel(point_idx_a, coord_a, a_F, point_idx_b, coord_b, b_F,
           W_fuse, b_fuse):
    qa, ga_pad, qb, gb_pad = _prep(point_idx_a, coord_a,
                                   point_idx_b, coord_b)

    idx, w = _knn(qa, ga_pad, qb, gb_pad)
    rows = _build_sc_gather()(b_F, idx.T.reshape(-1))
    return _combine(a_F, W_fuse, b_fuse, rows.reshape(TOPK, N_A, D), w)


# speedup vs baseline: 1.0803x; 1.0181x over previous
import functools

import jax
import jax.numpy as jnp
from jax import lax
from jax.experimental import pallas as pl
from jax.experimental.pallas import tpu as pltpu
from jax.experimental.pallas import tpu_sc as plsc

N_A = 16384
N_B = 16384
D = 256
TOPK = 3
FULL_SCALE = 128
R = 0.5

BLK_A = 1024
KBLK_A = 512
KGRID_A = N_A // KBLK_A
BLK_B = 2048
N_BLK_B = N_B // BLK_B
GRID_A = N_A // BLK_A
BIG = (1 << 30) + (1 << 24)

SC_WORKERS = 32
SC_TOTAL = N_A * TOPK
SC_PER_W = SC_TOTAL // SC_WORKERS
SC_CHUNK = 128
SC_N_CHUNK = SC_PER_W // SC_CHUNK


def _knn_body(qa_ref, ga_ref, qb_ref, gb_ref, idx_ref, w_ref, seg_ref):
    qa = qa_ref[...]
    ga = jnp.min(ga_ref[...], axis=1, keepdims=True)
    an = jnp.sum(qa * qa, axis=1, keepdims=True)
    iota = (lax.broadcasted_iota(jnp.int32, (KBLK_A, BLK_B), 1)
            + (1 << 23))
    big = jnp.int32(BIG)

    @pl.when(pl.program_id(0) == 0)
    def _():
        gb = gb_ref[...]
        for g in range(5):
            cnt = jnp.sum((gb < g).astype(jnp.int32))
            seg_ref[g] = cnt // 8

    g_lo = jnp.min(ga)
    g_hi = jnp.max(ga)
    t_lo = seg_ref[g_lo] // BLK_B
    t_hi = (seg_ref[g_hi + 1] + BLK_B - 1) // BLK_B

    r0 = jnp.full((KBLK_A, 1), big, jnp.int32)

    def scan_block(t, carry):
        r0, r1, r2 = carry
        base = t * BLK_B
        qb = qb_ref[:, pl.ds(base, BLK_B)]
        xy = jnp.dot(qa, qb, preferred_element_type=jnp.float32)
        bn = jnp.sum(qb * qb, axis=0, keepdims=True)
        d2 = (an + bn) - 2.0 * xy
        d2i = jnp.minimum(d2, 65536.0).astype(jnp.int32)
        keys = d2i * 16384 + (iota + base)

        kf = keys
        h = BLK_B // 2
        p0 = jnp.minimum(kf[:, 0:h], kf[:, h:BLK_B])
        p1 = jnp.maximum(kf[:, 0:h], kf[:, h:BLK_B])
        h //= 2
        a0, b0 = p0[:, 0:h], p0[:, h:2 * h]
        a1, b1 = p1[:, 0:h], p1[:, h:2 * h]
        t0 = jnp.minimum(a0, b0)
        x = jnp.maximum(a0, b0)
        y = jnp.minimum(a1, b1)
        t1 = jnp.minimum(x, y)
        t2 = jnp.maximum(x, y)

        def tri_merge(u0, u1, u2):
            h = u0.shape[1] // 2
            a0, b0 = u0[:, :h], u0[:, h:]
            a1, b1 = u1[:, :h], u1[:, h:]
            a2, b2 = u2[:, :h], u2[:, h:]
            m0 = jnp.minimum(a0, b0)
            hi = jnp.maximum(a0, b0)
            lo = jnp.minimum(a1, b1)
            m1 = jnp.minimum(hi, lo)
            tt = jnp.maximum(hi, lo)
            m2 = jnp.minimum(tt, jnp.minimum(a2, b2))
            return m0, m1, m2

        while h > 128:
            t0, t1, t2 = tri_merge(t0, t1, t2)
            h //= 2

        ks = jnp.concatenate([t0, t1, t2], axis=1)
        for _ in range(TOPK):
            m = jnp.min(ks, axis=1, keepdims=True)
            ks = jnp.where(ks == m, big, ks)
            h0 = jnp.maximum(r0, m)
            r0 = jnp.minimum(r0, m)
            h1 = jnp.maximum(r1, h0)
            r1 = jnp.minimum(r1, h0)
            r2 = jnp.minimum(r2, h1)
        return r0, r1, r2

    r0, r1, r2 = lax.fori_loop(t_lo, t_hi, scan_block, (r0, r0, r0))

    rr = jnp.concatenate([r0, r1, r2], axis=1)
    d2f = ((rr >> 14) - 512).astype(jnp.float32)
    dist = jnp.sqrt(d2f) * (1.0 / FULL_SCALE)
    idx_ref[...] = rr & 16383
    w_ref[...] = jnp.maximum(0.0, R - dist)


def _knn(qa, ga_pad, qb, gb_pad):
    return pl.pallas_call(
        _knn_body,
        grid=(KGRID_A,),
        in_specs=[
            pl.BlockSpec((KBLK_A, 8), lambda i: (i, 0)),
            pl.BlockSpec((KBLK_A, 8), lambda i: (i, 0)),
            pl.BlockSpec((8, N_B), lambda i: (0, 0)),
            pl.BlockSpec((8, N_B), lambda i: (0, 0)),
        ],
        out_specs=[
            pl.BlockSpec((KBLK_A, TOPK), lambda i: (i, 0)),
            pl.BlockSpec((KBLK_A, TOPK), lambda i: (i, 0)),
        ],
        out_shape=[
            jax.ShapeDtypeStruct((N_A, TOPK), jnp.int32),
            jax.ShapeDtypeStruct((N_A, TOPK), jnp.float32),
        ],
        scratch_shapes=[pltpu.SMEM((8,), jnp.int32)],
    )(qa, ga_pad, qb, gb_pad)


@functools.cache
def _build_sc_gather():
    mesh = plsc.VectorSubcoreMesh(core_axis_name="c", subcore_axis_name="s")

    @functools.partial(
        pl.kernel,
        mesh=mesh,
        out_type=jax.ShapeDtypeStruct((SC_TOTAL, D), jnp.float32),
        scratch_types=[
            pltpu.VMEM((SC_PER_W,), jnp.int32),
            pltpu.VMEM((SC_CHUNK, D), jnp.float32),
            pltpu.VMEM((SC_CHUNK, D), jnp.float32),
            pltpu.SemaphoreType.DMA,
            pltpu.SemaphoreType.DMA,
            pltpu.SemaphoreType.DMA,
            pltpu.SemaphoreType.DMA,
        ],
    )
    def sc_gather(table_hbm, idx_hbm, out_hbm, idx_v, rv0, rv1,
                  gs0, gs1, ws0, ws1):
        wid = lax.axis_index("s") * 2 + lax.axis_index("c")
        base = wid * SC_PER_W
        pltpu.sync_copy(idx_hbm.at[pl.ds(base, SC_PER_W)], idx_v)
        bufs = (rv0, rv1)
        gsem = (gs0, gs1)
        wsem = (ws0, ws1)

        def start_gather(c):
            return pltpu.async_copy(
                table_hbm.at[idx_v.at[pl.ds(c * SC_CHUNK, SC_CHUNK)]],
                bufs[c % 2], gsem[c % 2])

        h_g = start_gather(0)
        h_w = [None] * SC_N_CHUNK
        for c in range(SC_N_CHUNK):
            if c + 1 < SC_N_CHUNK:
                if c >= 1:
                    h_w[c - 1].wait()
                h_g_next = start_gather(c + 1)
            h_g.wait()
            h_w[c] = pltpu.async_copy(
                bufs[c % 2],
                out_hbm.at[pl.ds(base + c * SC_CHUNK, SC_CHUNK)],
                wsem[c % 2])
            if c + 1 < SC_N_CHUNK:
                h_g = h_g_next
        h_w[SC_N_CHUNK - 2].wait()
        h_w[SC_N_CHUNK - 1].wait()

    return sc_gather


def _combine_body(aF_ref, W_ref, bfuse_ref, rows_ref, w_ref, out_ref):
    W1 = W_ref[0:D, :]
    W2 = W_ref[D:2 * D, :]
    W12 = W1 - W2
    aF = aF_ref[...]
    A = jnp.dot(aF, W2, preferred_element_type=jnp.float32) + bfuse_ref[...]
    acc = jnp.zeros((BLK_A, D), jnp.float32)
    for k in range(TOPK):
        mk = jnp.dot(rows_ref[k], W12, preferred_element_type=jnp.float32)
        acc = acc + jnp.maximum(A + mk, 0.0) * w_ref[:, k:k + 1]
    out_ref[:, 0:D] = aF
    out_ref[:, D:2 * D] = acc


def _combine(a_F, W_fuse, b_fuse, rows3, w):
    return pl.pallas_call(
        _combine_body,
        grid=(GRID_A,),
        in_specs=[
            pl.BlockSpec((BLK_A, D), lambda i: (i, 0)),
            pl.BlockSpec((2 * D, D), lambda i: (0, 0)),
            pl.BlockSpec((1, D), lambda i: (0, 0)),
            pl.BlockSpec((TOPK, BLK_A, D), lambda i: (0, i, 0)),
            pl.BlockSpec((BLK_A, TOPK), lambda i: (i, 0)),
        ],
        out_specs=pl.BlockSpec((BLK_A, 2 * D), lambda i: (i, 0)),
        out_shape=jax.ShapeDtypeStruct((N_A, 2 * D), jnp.float32),
    )(a_F, W_fuse, b_fuse.reshape(1, D), rows3, w)


def _prep(point_idx_a, coord_a, point_idx_b, coord_b):
    ca = coord_a.astype(jnp.float32)
    cb = coord_b.astype(jnp.float32)
    pa = (point_idx_a.astype(jnp.float32) * 221.0)[:, None]
    pb = (point_idx_b.astype(jnp.float32) * 221.0)[:, None]
    qa = jnp.pad(jnp.concatenate([ca, pa], axis=1), ((0, 0), (0, 4)))
    qb = jnp.pad(jnp.concatenate([cb, pb], axis=1), ((0, 0), (0, 4))).T
    ga_pad = jnp.broadcast_to(point_idx_a[:, None].astype(jnp.int32),
                              (N_A, 8))
    gb_pad = jnp.broadcast_to(point_idx_b[None, :].astype(jnp.int32),
                              (8, N_B))
    return qa, ga_pad, qb, gb_pad


def kernel(point_idx_a, coord_a, a_F, point_idx_b, coord_b, b_F,
           W_fuse, b_fuse):
    qa, ga_pad, qb, gb_pad = _prep(point_idx_a, coord_a,
                                   point_idx_b, coord_b)

    idx, w = _knn(qa, ga_pad, qb, gb_pad)
    rows = _build_sc_gather()(b_F, idx.T.reshape(-1))
    return _combine(a_F, W_fuse, b_fuse, rows.reshape(TOPK, N_A, D), w)
